# baseline (speedup 1.0000x reference)
import functools

import jax
import jax.numpy as jnp
from jax import lax
from jax.experimental import pallas as pl
from jax.experimental.pallas import tpu as pltpu
from jax.experimental.pallas import tpu_sc as plsc

N = 10000
E = 320000
D = 128

NC = 2
NS = 16
NW = NC * NS
CHUNK = 64
NBUF = 4
G = 20
G0 = 4
G1 = 12
TG = NS * (G0 + G1)
EPADDED = TG * G * CHUNK
NPAD = 10240
STRIPE = NPAD // NS
ZCH = STRIPE // CHUNK

_mesh = plsc.VectorSubcoreMesh(
    core_axis_name="c", subcore_axis_name="s", num_cores=NC, num_subcores=NS
)


@functools.partial(
    pl.kernel,
    out_type=jax.ShapeDtypeStruct((NC, NPAD, D), jnp.float32),
    mesh=_mesh,
    scratch_types=[
        pltpu.VMEM((2, G, CHUNK), jnp.int32),
        pltpu.VMEM((2, G, CHUNK), jnp.int32),
        pltpu.VMEM((NBUF, CHUNK, D), jnp.float32),
        pltpu.VMEM_SHARED((NPAD, D), jnp.float32),
        pltpu.SemaphoreType.DMA((NBUF,)),
        pltpu.SemaphoreType.DMA((NBUF,)),
        pltpu.SemaphoreType.DMA((2,)),
    ],
)
def _sc_aggregate(x_hbm, ei_hbm, out_hbm, src_v, dst_v, rows, agg, gsems, ssems, isems):
    c = lax.axis_index("c")
    s = lax.axis_index("s")
    gbase = jnp.where(c == 0, s * G0, NS * G0 + s * G1)
    ngrp = jnp.where(c == 0, G0, G1)

    z16 = jnp.zeros((16,), jnp.float32)

    def _zero_body(t, carry):
        rows[0, t // (D // 16), pl.ds((t % (D // 16)) * 16, 16)] = z16
        return carry

    lax.fori_loop(0, CHUNK * (D // 16), _zero_body, 0)
    for k in range(ZCH):
        pltpu.sync_copy(rows.at[0], agg.at[pl.ds(s * STRIPE + k * CHUNK, CHUNK)])

    pltpu.sync_copy(ei_hbm.at[0, gbase], src_v.at[0])
    pltpu.sync_copy(ei_hbm.at[1, gbase], dst_v.at[0])
    plsc.subcore_barrier()

    def _group_body(g, carry):
        hg = g % 2
        hn = (g + 1) % 2

        @pl.when(g > 0)
        def _():
            pltpu.make_async_copy(
                ei_hbm.at[0, gbase + g], src_v.at[hg], isems.at[0]
            ).wait()
            pltpu.make_async_copy(
                ei_hbm.at[1, gbase + g], dst_v.at[hg], isems.at[1]
            ).wait()

        @pl.when(g + 1 < ngrp)
        def _():
            pltpu.async_copy(ei_hbm.at[0, gbase + g + 1], src_v.at[hn], isems.at[0])
            pltpu.async_copy(ei_hbm.at[1, gbase + g + 1], dst_v.at[hn], isems.at[1])

        def _issue(k, carry):
            b = k % NBUF
            pltpu.async_copy(x_hbm.at[src_v.at[hg, k]], rows.at[b], gsems.at[b])
            return carry

        def _wait_scatter(k, carry):
            b = k % NBUF
            pltpu.make_async_copy(
                rows.at[b], agg.at[dst_v.at[hg, k]], ssems.at[b]
            ).wait()
            return carry

        lax.fori_loop(0, 2, _issue, 0)

        def _chunk_body(k, carry):
            b = k % NBUF
            pltpu.make_async_copy(
                x_hbm.at[src_v.at[hg, k]], rows.at[b], gsems.at[b]
            ).wait()
            pltpu.async_copy(rows.at[b], agg.at[dst_v.at[hg, k]], ssems.at[b], add=True)

            @pl.when(k >= 2)
            def _():
                _wait_scatter(k - 2, 0)

            @pl.when(k + 2 < G)
            def _():
                _issue(k + 2, 0)

            return carry

        lax.fori_loop(0, G, _chunk_body, 0)
        lax.fori_loop(G - 2, G, _wait_scatter, 0)
        return carry

    lax.fori_loop(0, ngrp, _group_body, 0)
    plsc.subcore_barrier()

    pltpu.sync_copy(
        agg.at[pl.ds(s * STRIPE, STRIPE)],
        out_hbm.at[c, pl.ds(s * STRIPE, STRIPE)],
    )


def _dense_hidden_body(eps_ref, x_ref, agg_ref, w_ref, b_ref, g_ref, bt_ref, o_ref):
    h = x_ref[...] * (1.0 + eps_ref[0, 0]) + agg_ref[0, :N] + agg_ref[1, :N]
    h = jnp.dot(
        h, w_ref[...],
        precision=jax.lax.Precision.HIGHEST,
        preferred_element_type=jnp.float32,
    ) + b_ref[...]
    m = jnp.mean(h, axis=0, keepdims=True)
    v = jnp.mean((h - m) * (h - m), axis=0, keepdims=True)
    hn = g_ref[...] * (h - m) / jnp.sqrt(v + 1e-5) + bt_ref[...]
    o_ref[...] = jnp.maximum(hn, 0.0)


def _dense_final_body(eps_ref, x_ref, agg_ref, w_ref, b_ref, o_ref):
    h = x_ref[...] * (1.0 + eps_ref[0, 0]) + agg_ref[0, :N] + agg_ref[1, :N]
    h = jnp.dot(
        h, w_ref[...],
        precision=jax.lax.Precision.HIGHEST,
        preferred_element_type=jnp.float32,
    ) + b_ref[...]
    m = jnp.max(h, axis=-1, keepdims=True)
    e = h - m
    o_ref[...] = e - jnp.log(jnp.sum(jnp.exp(e), axis=-1, keepdims=True))


_dense_hidden = pl.pallas_call(
    _dense_hidden_body,
    out_shape=jax.ShapeDtypeStruct((N, D), jnp.float32),
)

_dense_final = pl.pallas_call(
    _dense_final_body,
    out_shape=jax.ShapeDtypeStruct((N, D), jnp.float32),
)


def kernel(x, edge_index, eps0, W0, b0, eps1, W1, b1, eps2, W2, b2, g0, bt0, g1, bt1):
    pad = jnp.stack(
        [
            jnp.zeros((EPADDED - E,), jnp.int32),
            jnp.full((EPADDED - E,), N, jnp.int32),
        ]
    )
    ei = jnp.concatenate([edge_index, pad], axis=1).reshape(2, TG, G, CHUNK)

    def row(a):
        return a.reshape(1, D)

    h = x
    agg = _sc_aggregate(h, ei)
    h = _dense_hidden(eps0.reshape(1, 1), h, agg, W0, row(b0), row(g0), row(bt0))
    agg = _sc_aggregate(h, ei)
    h = _dense_hidden(eps1.reshape(1, 1), h, agg, W1, row(b1), row(g1), row(bt1))
    agg = _sc_aggregate(h, ei)
    return _dense_final(eps2.reshape(1, 1), h, agg, W2, row(b2))


# speedup vs baseline: 1.0872x; 1.0872x over previous
import functools

import jax
import jax.numpy as jnp
from jax import lax
from jax.experimental import pallas as pl
from jax.experimental.pallas import tpu as pltpu
from jax.experimental.pallas import tpu_sc as plsc

N = 10000
E = 320000
D = 128

NC = 2
NS = 16
NW = NC * NS
CHUNK = 128
G = 10
NGRP = 8
NCHUNK = NGRP * G
EPT = NCHUNK * CHUNK
EPADDED = NW * EPT
NPAD = 10240
STRIPE = NPAD // NS
ZCH = STRIPE // CHUNK

_mesh = plsc.VectorSubcoreMesh(
    core_axis_name="c", subcore_axis_name="s", num_cores=NC, num_subcores=NS
)


@functools.partial(
    pl.kernel,
    out_type=jax.ShapeDtypeStruct((NC, NPAD, D), jnp.float32),
    mesh=_mesh,
    scratch_types=[
        pltpu.VMEM((2, G, CHUNK), jnp.int32),
        pltpu.VMEM((2, G, CHUNK), jnp.int32),
        pltpu.VMEM((2, CHUNK, D), jnp.float32),
        pltpu.VMEM_SHARED((NPAD, D), jnp.float32),
        pltpu.SemaphoreType.DMA((2,)),
        pltpu.SemaphoreType.DMA((2,)),
    ],
)
def _sc_aggregate(x_hbm, ei_hbm, out_hbm, src_v, dst_v, rows, agg, sems, isems):
    c = lax.axis_index("c")
    s = lax.axis_index("s")
    wid = c * NS + s

    pltpu.async_copy(ei_hbm.at[0, wid, 0], src_v.at[0], isems.at[0])
    pltpu.async_copy(ei_hbm.at[1, wid, 0], dst_v.at[0], isems.at[1])

    z16 = jnp.zeros((16,), jnp.float32)

    def _zero_body(t, carry):
        rows[0, t // (D // 16), pl.ds((t % (D // 16)) * 16, 16)] = z16
        return carry

    lax.fori_loop(0, CHUNK * (D // 16), _zero_body, 0)

    def _zissue(k, carry):
        pltpu.async_copy(
            rows.at[0], agg.at[pl.ds(s * STRIPE + k * CHUNK, CHUNK)], sems.at[k % 2]
        )
        return carry

    def _zwait(k, carry):
        pltpu.make_async_copy(
            rows.at[0], agg.at[pl.ds(s * STRIPE + k * CHUNK, CHUNK)], sems.at[k % 2]
        ).wait()
        return carry

    lax.fori_loop(0, ZCH, _zissue, 0)
    lax.fori_loop(0, ZCH, _zwait, 0)
    pltpu.make_async_copy(ei_hbm.at[0, wid, 0], src_v.at[0], isems.at[0]).wait()
    pltpu.make_async_copy(ei_hbm.at[1, wid, 0], dst_v.at[0], isems.at[1]).wait()
    plsc.subcore_barrier()

    def _group_body(g, carry):
        hg = g % 2
        hn = (g + 1) % 2

        @pl.when(g > 0)
        def _():
            pltpu.make_async_copy(ei_hbm.at[0, wid, g], src_v.at[hg], isems.at[0]).wait()
            pltpu.make_async_copy(ei_hbm.at[1, wid, g], dst_v.at[hg], isems.at[1]).wait()

        @pl.when(g + 1 < NGRP)
        def _():
            pltpu.async_copy(ei_hbm.at[0, wid, g + 1], src_v.at[hn], isems.at[0])
            pltpu.async_copy(ei_hbm.at[1, wid, g + 1], dst_v.at[hn], isems.at[1])

        def _issue(k, carry):
            b = k % 2
            pltpu.async_copy(x_hbm.at[src_v.at[hg, k]], rows.at[b], sems.at[b])
            return carry

        lax.fori_loop(0, 2, _issue, 0)

        def _chunk_body(k, carry):
            b = k % 2
            pltpu.make_async_copy(
                x_hbm.at[src_v.at[hg, k]], rows.at[b], sems.at[b]
            ).wait()
            pltpu.sync_copy(rows.at[b], agg.at[dst_v.at[hg, k]], add=True)

            @pl.when(k + 2 < G)
            def _():
                _issue(k + 2, 0)

            return carry

        lax.fori_loop(0, G, _chunk_body, 0)
        return carry

    lax.fori_loop(0, NGRP, _group_body, 0)
    plsc.subcore_barrier()

    pltpu.sync_copy(
        agg.at[pl.ds(s * STRIPE, STRIPE)],
        out_hbm.at[c, pl.ds(s * STRIPE, STRIPE)],
    )


def _dense_hidden_body(eps_ref, x_ref, agg_ref, w_ref, b_ref, g_ref, bt_ref, o_ref):
    h = x_ref[...] * (1.0 + eps_ref[0, 0]) + agg_ref[0, :N] + agg_ref[1, :N]
    h = jnp.dot(
        h, w_ref[...],
        precision=jax.lax.Precision.HIGHEST,
        preferred_element_type=jnp.float32,
    ) + b_ref[...]
    m = jnp.mean(h, axis=0, keepdims=True)
    v = jnp.mean((h - m) * (h - m), axis=0, keepdims=True)
    hn = g_ref[...] * (h - m) / jnp.sqrt(v + 1e-5) + bt_ref[...]
    o_ref[...] = jnp.maximum(hn, 0.0)


def _dense_final_body(eps_ref, x_ref, agg_ref, w_ref, b_ref, o_ref):
    h = x_ref[...] * (1.0 + eps_ref[0, 0]) + agg_ref[0, :N] + agg_ref[1, :N]
    h = jnp.dot(
        h, w_ref[...],
        precision=jax.lax.Precision.HIGHEST,
        preferred_element_type=jnp.float32,
    ) + b_ref[...]
    m = jnp.max(h, axis=-1, keepdims=True)
    e = h - m
    o_ref[...] = e - jnp.log(jnp.sum(jnp.exp(e), axis=-1, keepdims=True))


_dense_hidden = pl.pallas_call(
    _dense_hidden_body,
    out_shape=jax.ShapeDtypeStruct((N, D), jnp.float32),
)

_dense_final = pl.pallas_call(
    _dense_final_body,
    out_shape=jax.ShapeDtypeStruct((N, D), jnp.float32),
)


def kernel(x, edge_index, eps0, W0, b0, eps1, W1, b1, eps2, W2, b2, g0, bt0, g1, bt1):
    pad = jnp.stack(
        [
            jnp.zeros((EPADDED - E,), jnp.int32),
            jnp.full((EPADDED - E,), N, jnp.int32),
        ]
    )
    ei = jnp.concatenate([edge_index, pad], axis=1).reshape(2, NW, NGRP, G, CHUNK)

    def row(a):
        return a.reshape(1, D)

    h = x
    agg = _sc_aggregate(h, ei)
    h = _dense_hidden(eps0.reshape(1, 1), h, agg, W0, row(b0), row(g0), row(bt0))
    agg = _sc_aggregate(h, ei)
    h = _dense_hidden(eps1.reshape(1, 1), h, agg, W1, row(b1), row(g1), row(bt1))
    agg = _sc_aggregate(h, ei)
    return _dense_final(eps2.reshape(1, 1), h, agg, W2, row(b2))
